# baseline (speedup 1.0000x reference)
import functools

import jax
import jax.numpy as jnp
from jax import lax
from jax.experimental import pallas as pl
from jax.experimental.pallas import tpu as pltpu
from jax.experimental.pallas import tpu_sc as plsc

B = 16384
D = 512
L = 16
NC = 2
NS = 16
NW = NC * NS
BPW = B // NW
CH = 32
NCH = BPW // CH
NACC = 4

_mesh = plsc.VectorSubcoreMesh(core_axis_name="c", subcore_axis_name="s")


@functools.partial(
    pl.kernel,
    mesh=_mesh,
    out_type=jax.ShapeDtypeStruct((NW, L), jnp.float32),
    scratch_types=[
        pltpu.VMEM((BPW,), jnp.int32),
        pltpu.VMEM((CH, D), jnp.float32),
        pltpu.VMEM((CH, D), jnp.float32),
        pltpu.VMEM((L,), jnp.float32),
        pltpu.SemaphoreType.DMA,
        pltpu.SemaphoreType.DMA,
    ],
)
def _sc_partials(features_hbm, target_hbm, center_hbm, out_hbm,
                 idx_v, fbuf, cbuf, accv, semf, semc):
    wid = lax.axis_index("s") * NC + lax.axis_index("c")
    base = wid * BPW
    pltpu.sync_copy(target_hbm.at[pl.ds(base, BPW)], idx_v)

    accs = [jnp.zeros((L,), jnp.float32) for _ in range(NACC)]
    for g in range(NCH):
        cpf = pltpu.async_copy(
            features_hbm.at[pl.ds(base + g * CH, CH)], fbuf, semf)
        cpc = pltpu.async_copy(
            center_hbm.at[idx_v.at[pl.ds(g * CH, CH)]], cbuf, semc)
        cpf.wait()
        cpc.wait()

        def row_body(r, accs):
            accs = list(accs)
            for v in range(D // L):
                df = fbuf[r, pl.ds(v * L, L)] - cbuf[r, pl.ds(v * L, L)]
                accs[v % NACC] = accs[v % NACC] + df * df
            return tuple(accs)

        accs = list(lax.fori_loop(0, CH, row_body, tuple(accs)))

    total = accs[0]
    for a in accs[1:]:
        total = total + a
    accv[...] = total
    pltpu.sync_copy(accv, out_hbm.at[wid])


def kernel(features, target, center):
    partials = _sc_partials(features, target.astype(jnp.int32), center)
    return jnp.sum(partials) * (1.0 / B)
